# baseline (speedup 1.0000x reference)
import functools

import jax
import jax.numpy as jnp
from jax import lax
from jax.experimental import pallas as pl
from jax.experimental.pallas import tpu as pltpu
from jax.experimental.pallas import tpu_sc as plsc

D = 64
SCALE = 8.0
NC = 2
NS = 16
NW = NC * NS
T = 200
BR = 128
NBUF = 2

_mesh = plsc.VectorSubcoreMesh(core_axis_name="c", subcore_axis_name="s")


@functools.partial(
    pl.kernel,
    out_type=jax.ShapeDtypeStruct((T, 8, NW, 8, BR), jnp.float32),
    mesh=_mesh,
    scratch_types=[
        pltpu.VMEM((T // 8, 8, BR), jnp.int32),
        pltpu.VMEM((NBUF, BR, D), jnp.float32),
        pltpu.VMEM((NBUF, 8, 8, BR), jnp.float32),
        pltpu.SemaphoreType.DMA((NBUF,)),
        pltpu.SemaphoreType.DMA((NBUF,)),
    ],
    compiler_params=pltpu.CompilerParams(use_tc_tiling_on_sc=False,
                                         needs_layout_passes=False),
)
def _emb_lookup(table_hbm, x4_hbm, out_hbm, idx_v, rows_v, blk_v, gsem, ssem):
    w = lax.axis_index("s") * NC + lax.axis_index("c")
    pltpu.sync_copy(x4_hbm.at[:, w], idx_v)

    def idx_slice(t):
        return idx_v.at[lax.div(t, 8), lax.rem(t, 8)]

    def gfire(t, b):
        pltpu.async_copy(table_hbm.at[idx_slice(t)], rows_v.at[b], gsem.at[b])

    def gwait(t, b):
        pltpu.make_async_copy(table_hbm.at[idx_slice(t)], rows_v.at[b],
                              gsem.at[b]).wait()

    def sfire(t, b):
        pltpu.async_copy(blk_v.at[b], out_hbm.at[t, :, w], ssem.at[b])

    def swait(t, b):
        pltpu.make_async_copy(blk_v.at[b], out_hbm.at[t, :, w],
                              ssem.at[b]).wait()

    iota = lax.iota(jnp.int32, 16)
    row_ids = [iota + (k * 16) for k in range(BR // 16)]

    def transpose_scale(b):
        rows = rows_v.at[b]
        for d in range(D):
            col = jnp.full((16,), d, jnp.int32)
            for k in range(BR // 16):
                v = plsc.load_gather(rows, [row_ids[k], col])
                blk_v[b, d // 8, d % 8, pl.ds(k * 16, 16)] = v * SCALE

    for b in range(NBUF):
        gfire(b, b)

    def round_body(g, carry):
        t0 = g * NBUF
        for b in range(NBUF):
            t = t0 + b
            gwait(t, b)

            @pl.when(t >= NBUF)
            def _():
                swait(t - NBUF, b)

            transpose_scale(b)
            sfire(t, b)

            @pl.when(t + NBUF < T)
            def _():
                gfire(t + NBUF, b)

        return carry

    lax.fori_loop(0, T // NBUF, round_body, 0)
    for b in range(NBUF):
        swait(T - NBUF + b, b)


def kernel(x, table):
    x4 = x.astype(jnp.int32).reshape(NW, BR, T // 8, 8).transpose(2, 0, 3, 1)
    out5 = _emb_lookup(table, x4)
    return out5.transpose(2, 4, 0, 1, 3).reshape(x.shape[0], x.shape[1], D)


# speedup vs baseline: 6.1724x; 6.1724x over previous
import functools

import jax
import jax.numpy as jnp
from jax import lax
from jax.experimental import pallas as pl
from jax.experimental.pallas import tpu as pltpu
from jax.experimental.pallas import tpu_sc as plsc

D = 64
SCALE = 8.0
NC = 2
NS = 16
NW = NC * NS
T = 200
BR = 128
NBUF = 5

_mesh = plsc.VectorSubcoreMesh(core_axis_name="c", subcore_axis_name="s")

NBLK = 7812
NROWS1 = NBLK * BR
RNB = 4


@functools.partial(
    pl.kernel,
    out_type=jax.ShapeDtypeStruct((62500, 8, BR), jnp.float32),
    mesh=_mesh,
    scratch_types=[
        pltpu.VMEM((RNB, D, BR), jnp.float32),
        pltpu.VMEM((D * (BR + 1),), jnp.float32),
        pltpu.VMEM((RNB, 8, 8, BR), jnp.float32),
        pltpu.SemaphoreType.DMA((RNB,)),
        pltpu.SemaphoreType.DMA((RNB,)),
    ],
    compiler_params=pltpu.CompilerParams(use_tc_tiling_on_sc=True,
                                         needs_layout_passes=False),
)
def _relayout(tt_hbm, out_hbm, slab_v, rp_v, blk_v, gsem, ssem):
    w = lax.axis_index("s") * NC + lax.axis_index("c")
    P2 = BR + 1
    iota = lax.iota(jnp.int32, 16)
    base_ids = [iota * P2 + (16 * P2) * k for k in range(D // 16)]

    def blk_id(j):
        return w + NW * j

    def gfire(j, b):
        pltpu.async_copy(tt_hbm.at[:, pl.ds(blk_id(j) * BR, BR)],
                         slab_v.at[b], gsem.at[b])

    def gwait(j, b):
        pltpu.make_async_copy(tt_hbm.at[:, pl.ds(blk_id(j) * BR, BR)],
                              slab_v.at[b], gsem.at[b]).wait()

    def sfire(j, b):
        pltpu.async_copy(blk_v.at[b], out_hbm.at[pl.ds(blk_id(j) * 8, 8)],
                         ssem.at[b])

    def swait(j, b):
        pltpu.make_async_copy(blk_v.at[b],
                              out_hbm.at[pl.ds(blk_id(j) * 8, 8)],
                              ssem.at[b]).wait()

    def transpose(b):
        @plsc.parallel_loop(0, D, unroll=2)
        def _(r):
            for k in range(BR // 16):
                rp_v[pl.ds(r * P2 + k * 16, 16)] = slab_v[b, r,
                                                          pl.ds(k * 16, 16)]

        @plsc.parallel_loop(0, BR, unroll=2)
        def _(i):
            vs = [plsc.load_gather(rp_v, [base_ids[k] + i])
                  for k in range(D // 16)]
            s1 = lax.div(i, 16)
            s2 = lax.rem(lax.div(i, 2), 8)
            c0 = lax.rem(i, 2) * D
            for k in range(D // 16):
                blk_v[b, s1, s2, pl.ds(c0 + k * 16, 16)] = vs[k]

    JFULL = NBLK // NW
    NTAIL = NBLK - JFULL * NW

    for b in range(RNB):
        gfire(b, b)

    def round_body(g, carry):
        j0 = g * RNB
        for b in range(RNB):
            j = j0 + b
            gwait(j, b)

            @pl.when(j >= RNB)
            def _():
                swait(j - RNB, b)

            transpose(b)
            sfire(j, b)

            @pl.when(blk_id(j + RNB) < NBLK)
            def _():
                gfire(j + RNB, b)

        return carry

    lax.fori_loop(0, JFULL // RNB, round_body, 0)

    @pl.when(w < NTAIL)
    def _():
        swait(JFULL - RNB, 0)
        gwait(JFULL, 0)
        transpose(0)
        sfire(JFULL, 0)

    @pl.when(w < NTAIL)
    def _():
        swait(JFULL, 0)

    @pl.when(w >= NTAIL)
    def _():
        swait(JFULL - RNB, 0)

    for b in range(1, RNB):
        swait(JFULL - RNB + b, b)


@functools.partial(
    pl.kernel,
    out_type=jax.ShapeDtypeStruct((T, 8, NW, 8 * BR), jnp.float32),
    mesh=_mesh,
    scratch_types=[
        pltpu.VMEM((T // 8, 8, BR), jnp.int32),
        pltpu.VMEM((NBUF, BR, D), jnp.float32),
        pltpu.VMEM((BR * (D + 1),), jnp.float32),
        pltpu.VMEM((NBUF, 8, 8 * BR), jnp.float32),
        pltpu.VMEM((D, D), jnp.float32),
        pltpu.SemaphoreType.DMA((NBUF,)),
        pltpu.SemaphoreType.DMA((NBUF,)),
    ],
    compiler_params=pltpu.CompilerParams(use_tc_tiling_on_sc=False,
                                         needs_layout_passes=False),
)
def _emb_lookup(table_hbm, x4_hbm, aux_hbm, out_hbm, idx_v, rows_v, rp_v,
                blk_v, aux_v, gsem, ssem):
    w = lax.axis_index("s") * NC + lax.axis_index("c")
    pltpu.sync_copy(x4_hbm.at[:, w], idx_v)
    pltpu.sync_copy(aux_hbm, aux_v)

    def idx_slice(t):
        return idx_v.at[lax.div(t, 8), lax.rem(t, 8)]

    def gfire(t, b):
        pltpu.async_copy(table_hbm.at[idx_slice(t)], rows_v.at[b], gsem.at[b])

    def gwait(t, b):
        pltpu.make_async_copy(table_hbm.at[idx_slice(t)], rows_v.at[b],
                              gsem.at[b]).wait()

    def sfire(t, b):
        pltpu.async_copy(blk_v.at[b], out_hbm.at[t, :, w], ssem.at[b])

    def swait(t, b):
        pltpu.make_async_copy(blk_v.at[b], out_hbm.at[t, :, w],
                              ssem.at[b]).wait()

    iota = lax.iota(jnp.int32, 16)
    P = D + 1
    base_ids = [iota * P + (16 * P) * k for k in range(BR // 16)]

    def transpose_scale(t, b):
        @plsc.parallel_loop(0, BR, unroll=2)
        def _(r):
            for k in range(D // 16):
                rp_v[pl.ds(r * P + k * 16, 16)] = rows_v[b, r,
                                                         pl.ds(k * 16, 16)]

        @plsc.parallel_loop(0, D, unroll=2)
        def _(d):
            vs = [plsc.load_gather(rp_v, [base_ids[k] + d])
                  for k in range(BR // 16)]
            dt = lax.div(d, 8)
            off = lax.rem(d, 8) * BR
            for k in range(BR // 16):
                blk_v[b, dt, pl.ds(off + k * 16, 16)] = vs[k] * SCALE

        tq, tr = lax.div(t, 8), lax.rem(t, 8)
        ivs = [idx_v[tq, tr, pl.ds(k * 16, 16)] for k in range(BR // 16)]
        ms = [iv >= NROWS1 for iv in ivs]
        mm = ms[0]
        for m in ms[1:]:
            mm = mm | m

        @pl.when(jnp.any(mm))
        def _():
            @plsc.parallel_loop(0, D)
            def _(d):
                dt = lax.div(d, 8)
                off = lax.rem(d, 8) * BR
                dcol = jnp.zeros((16,), jnp.int32) + d
                for k in range(BR // 16):
                    vals = plsc.load_gather(
                        aux_v, [ivs[k] - NROWS1, dcol], mask=ms[k]) * SCALE
                    plsc.store_scatter(
                        blk_v.at[b],
                        [jnp.zeros((16,), jnp.int32) + dt,
                         iota + (off + k * 16)],
                        vals, mask=ms[k])

    for b in range(NBUF):
        gfire(b, b)

    def round_body(g, carry):
        t0 = g * NBUF
        for b in range(NBUF):
            t = t0 + b
            gwait(t, b)

            @pl.when(t >= NBUF)
            def _():
                swait(t - NBUF, b)

            transpose_scale(t, b)
            sfire(t, b)

            @pl.when(t + NBUF < T)
            def _():
                gfire(t + NBUF, b)

        return carry

    lax.fori_loop(0, T // NBUF, round_body, 0)
    for b in range(NBUF):
        swait(T - NBUF + b, b)


def kernel(x, table):
    x4 = x.astype(jnp.int32).reshape(NW, BR, T // 8, 8).transpose(2, 0, 3, 1)
    table_lin = _relayout(table.T).reshape(1000000, D)
    aux = table[NROWS1:]
    out4 = _emb_lookup(table_lin, x4, aux)
    out5 = out4.reshape(T, 8, NW, 8, BR)
    return out5.transpose(2, 4, 0, 1, 3).reshape(x.shape[0], x.shape[1], D)


# speedup vs baseline: 6.2119x; 1.0064x over previous
import functools

import jax
import jax.numpy as jnp
from jax import lax
from jax.experimental import pallas as pl
from jax.experimental.pallas import tpu as pltpu
from jax.experimental.pallas import tpu_sc as plsc

D = 64
SCALE = 8.0
NC = 2
NS = 16
NW = NC * NS
T = 200
BR = 128
NBUF = 4

_mesh = plsc.VectorSubcoreMesh(core_axis_name="c", subcore_axis_name="s")

NBLK = 7812
NROWS1 = NBLK * BR
RNB = 4


@functools.partial(
    pl.kernel,
    out_type=jax.ShapeDtypeStruct((62500, 8, BR), jnp.float32),
    mesh=_mesh,
    scratch_types=[
        pltpu.VMEM((RNB, D, BR), jnp.float32),
        pltpu.VMEM((D * (BR + 1),), jnp.float32),
        pltpu.VMEM((RNB, 8, 8, BR), jnp.float32),
        pltpu.SemaphoreType.DMA((RNB,)),
        pltpu.SemaphoreType.DMA((RNB,)),
    ],
    compiler_params=pltpu.CompilerParams(use_tc_tiling_on_sc=True,
                                         needs_layout_passes=False),
)
def _relayout(tt_hbm, out_hbm, slab_v, rp_v, blk_v, gsem, ssem):
    w = lax.axis_index("s") * NC + lax.axis_index("c")
    P2 = BR + 1
    iota = lax.iota(jnp.int32, 16)
    base_ids = [iota * P2 + (16 * P2) * k for k in range(D // 16)]

    def blk_id(j):
        return w + NW * j

    def gfire(j, b):
        pltpu.async_copy(tt_hbm.at[:, pl.ds(blk_id(j) * BR, BR)],
                         slab_v.at[b], gsem.at[b])

    def gwait(j, b):
        pltpu.make_async_copy(tt_hbm.at[:, pl.ds(blk_id(j) * BR, BR)],
                              slab_v.at[b], gsem.at[b]).wait()

    def sfire(j, b):
        pltpu.async_copy(blk_v.at[b], out_hbm.at[pl.ds(blk_id(j) * 8, 8)],
                         ssem.at[b])

    def swait(j, b):
        pltpu.make_async_copy(blk_v.at[b],
                              out_hbm.at[pl.ds(blk_id(j) * 8, 8)],
                              ssem.at[b]).wait()

    def transpose(b):
        @plsc.parallel_loop(0, D, unroll=2)
        def _(r):
            for k in range(BR // 16):
                rp_v[pl.ds(r * P2 + k * 16, 16)] = slab_v[b, r,
                                                          pl.ds(k * 16, 16)]

        @plsc.parallel_loop(0, BR, unroll=2)
        def _(i):
            vs = [plsc.load_gather(rp_v, [base_ids[k] + i])
                  for k in range(D // 16)]
            s1 = lax.div(i, 16)
            s2 = lax.rem(lax.div(i, 2), 8)
            c0 = lax.rem(i, 2) * D
            for k in range(D // 16):
                blk_v[b, s1, s2, pl.ds(c0 + k * 16, 16)] = vs[k]

    JFULL = NBLK // NW
    NTAIL = NBLK - JFULL * NW

    for b in range(RNB):
        gfire(b, b)

    def round_body(g, carry):
        j0 = g * RNB
        for b in range(RNB):
            j = j0 + b
            gwait(j, b)

            @pl.when(j >= RNB)
            def _():
                swait(j - RNB, b)

            transpose(b)
            sfire(j, b)

            @pl.when(blk_id(j + RNB) < NBLK)
            def _():
                gfire(j + RNB, b)

        return carry

    lax.fori_loop(0, JFULL // RNB, round_body, 0)

    @pl.when(w < NTAIL)
    def _():
        swait(JFULL - RNB, 0)
        gwait(JFULL, 0)
        transpose(0)
        sfire(JFULL, 0)

    @pl.when(w < NTAIL)
    def _():
        swait(JFULL, 0)

    @pl.when(w >= NTAIL)
    def _():
        swait(JFULL - RNB, 0)

    for b in range(1, RNB):
        swait(JFULL - RNB + b, b)


@functools.partial(
    pl.kernel,
    out_type=jax.ShapeDtypeStruct((T, 8, NW, 8 * BR), jnp.float32),
    mesh=_mesh,
    scratch_types=[
        pltpu.VMEM((T // 8, 8, BR), jnp.int32),
        pltpu.VMEM((NBUF, BR, D), jnp.float32),
        pltpu.VMEM((BR * (D + 1),), jnp.float32),
        pltpu.VMEM((NBUF, 8, 8 * BR), jnp.float32),
        pltpu.VMEM((D, D), jnp.float32),
        pltpu.SemaphoreType.DMA((NBUF,)),
        pltpu.SemaphoreType.DMA((NBUF,)),
    ],
    compiler_params=pltpu.CompilerParams(use_tc_tiling_on_sc=False,
                                         needs_layout_passes=False),
)
def _emb_lookup(table_hbm, x4_hbm, aux_hbm, out_hbm, idx_v, rows_v, rp_v,
                blk_v, aux_v, gsem, ssem):
    w = lax.axis_index("s") * NC + lax.axis_index("c")
    pltpu.sync_copy(x4_hbm.at[:, w], idx_v)
    pltpu.sync_copy(aux_hbm, aux_v)

    def idx_slice(t):
        return idx_v.at[lax.div(t, 8), lax.rem(t, 8)]

    def gfire(t, b):
        pltpu.async_copy(table_hbm.at[idx_slice(t)], rows_v.at[b], gsem.at[b])

    def gwait(t, b):
        pltpu.make_async_copy(table_hbm.at[idx_slice(t)], rows_v.at[b],
                              gsem.at[b]).wait()

    def sfire(t, b):
        pltpu.async_copy(blk_v.at[b], out_hbm.at[t, :, w], ssem.at[b])

    def swait(t, b):
        pltpu.make_async_copy(blk_v.at[b], out_hbm.at[t, :, w],
                              ssem.at[b]).wait()

    iota = lax.iota(jnp.int32, 16)
    P = D + 1
    base_ids = [iota * P + (16 * P) * k for k in range(BR // 16)]

    def transpose_scale(t, b):
        @plsc.parallel_loop(0, BR, unroll=2)
        def _(r):
            for k in range(D // 16):
                rp_v[pl.ds(r * P + k * 16, 16)] = rows_v[b, r,
                                                         pl.ds(k * 16, 16)]

        @plsc.parallel_loop(0, D, unroll=2)
        def _(d):
            vs = [plsc.load_gather(rp_v, [base_ids[k] + d])
                  for k in range(BR // 16)]
            dt = lax.div(d, 8)
            off = lax.rem(d, 8) * BR
            for k in range(BR // 16):
                blk_v[b, dt, pl.ds(off + k * 16, 16)] = vs[k] * SCALE

        tq, tr = lax.div(t, 8), lax.rem(t, 8)
        ivs = [idx_v[tq, tr, pl.ds(k * 16, 16)] for k in range(BR // 16)]
        ms = [iv >= NROWS1 for iv in ivs]
        mm = ms[0]
        for m in ms[1:]:
            mm = mm | m

        @pl.when(jnp.any(mm))
        def _():
            @plsc.parallel_loop(0, D)
            def _(d):
                dt = lax.div(d, 8)
                off = lax.rem(d, 8) * BR
                dcol = jnp.zeros((16,), jnp.int32) + d
                for k in range(BR // 16):
                    vals = plsc.load_gather(
                        aux_v, [ivs[k] - NROWS1, dcol], mask=ms[k]) * SCALE
                    plsc.store_scatter(
                        blk_v.at[b],
                        [jnp.zeros((16,), jnp.int32) + dt,
                         iota + (off + k * 16)],
                        vals, mask=ms[k])

    for b in range(NBUF):
        gfire(b, b)

    def round_body(g, carry):
        t0 = g * NBUF
        for b in range(NBUF):
            t = t0 + b
            gwait(t, b)

            @pl.when(t >= NBUF)
            def _():
                swait(t - NBUF, b)

            transpose_scale(t, b)
            sfire(t, b)

            @pl.when(t + NBUF < T)
            def _():
                gfire(t + NBUF, b)

        return carry

    lax.fori_loop(0, T // NBUF, round_body, 0)
    for b in range(NBUF):
        swait(T - NBUF + b, b)


def kernel(x, table):
    x4 = x.astype(jnp.int32).reshape(NW, BR, T // 8, 8).transpose(2, 0, 3, 1)
    table_lin = _relayout(table.T).reshape(1000000, D)
    aux = table[NROWS1:]
    out4 = _emb_lookup(table_lin, x4, aux)
    out5 = out4.reshape(T, 8, NW, 8, BR)
    return out5.transpose(2, 4, 0, 1, 3).reshape(x.shape[0], x.shape[1], D)
